# Initial kernel scaffold; baseline (speedup 1.0000x reference)
import jax
import jax.numpy as jnp
from jax.experimental import pallas as pl


def kernel(x, edge_index, edge_weight, W_rel1, b_rel1, W_root1, W_rel2, b_rel2, W_root2):
    raise NotImplementedError("write your pallas kernel here")


# speedup vs baseline: 26.3486x; 26.3486x over previous
import functools

import jax
import jax.numpy as jnp
from jax import lax
from jax.experimental import pallas as pl
from jax.experimental.pallas import tpu as pltpu
from jax.experimental.pallas import tpu_sc as plsc

N = 10000
E = 320000
DIN = 128
DH = 16

NC = 2
NS = 16
NW = NC * NS
EPW = E // NW
CH = 80
NCHUNK = EPW // CH

RPT = 624


def _proj1_body(x_ref, w_ref, o1_ref, o2_ref):
    r = jnp.dot(x_ref[...], w_ref[...], preferred_element_type=jnp.float32)
    o1_ref[...] = r[:, :DH]
    o2_ref[...] = r[:, DH:]


def _proj1(x, w1t):
    return pl.pallas_call(
        _proj1_body,
        grid=(10,),
        in_specs=[
            pl.BlockSpec((1000, DIN), lambda i: (i, 0)),
            pl.BlockSpec((DIN, 2 * DH), lambda i: (0, 0)),
        ],
        out_specs=[
            pl.BlockSpec((1000, DH), lambda i: (i, 0)),
            pl.BlockSpec((1000, DH), lambda i: (i, 0)),
        ],
        out_shape=[
            jax.ShapeDtypeStruct((N, DH), jnp.float32),
            jax.ShapeDtypeStruct((N, DH), jnp.float32),
        ],
    )(x, w1t)


def _seg16_body(xr_hbm, src_hbm, dst_hbm, w_hbm, out_hbm,
                src_v, dst_v, w_v, rows_v, buf_v, acc_sh, gsem, ssem):
    cid = lax.axis_index("c")
    sid = lax.axis_index("s")
    wid = cid * NS + sid

    pltpu.sync_copy(src_hbm.at[wid], src_v)
    pltpu.sync_copy(dst_hbm.at[wid], dst_v)
    pltpu.sync_copy(w_hbm.at[wid], w_v)

    pltpu.async_copy(xr_hbm.at[src_v.at[0]], rows_v.at[0], gsem.at[0])
    pltpu.async_copy(xr_hbm.at[src_v.at[1]], rows_v.at[1], gsem.at[1])

    def _zero(i, _):
        buf_v[i, :] = jnp.zeros((DH,), jnp.float32)
        return 0
    lax.fori_loop(0, RPT, _zero, 0)
    pltpu.sync_copy(buf_v, acc_sh.at[pl.ds(sid * RPT, RPT)])

    @pl.when(sid == NS - 1)
    def _():
        pltpu.sync_copy(buf_v.at[pl.ds(0, 16)], acc_sh.at[pl.ds(NS * RPT, 16)])

    plsc.subcore_barrier()

    def _scale(rows, c):
        def grp(g, _):
            wvec = w_v[c, pl.ds(g * 16, 16)]
            for u in range(16):
                i = g * 16 + u
                rows[i, :] = rows[i, :] * wvec[u]
            return 0
        lax.fori_loop(0, CH // 16, grp, 0)

    cl = NCHUNK - 1

    def _chunk(c, b):
        b2 = (b + 2) % 4
        @pl.when(c >= 2)
        def _():
            pltpu.make_async_copy(
                rows_v.at[b2], acc_sh.at[dst_v.at[c - 2]], ssem.at[b2]).wait()
        @pl.when(c + 2 <= cl)
        def _():
            pltpu.async_copy(
                xr_hbm.at[src_v.at[c + 2]], rows_v.at[b2], gsem.at[b2])
        pltpu.make_async_copy(
            xr_hbm.at[src_v.at[c]], rows_v.at[b], gsem.at[b]).wait()
        _scale(rows_v.at[b], c)
        pltpu.async_copy(rows_v.at[b], acc_sh.at[dst_v.at[c]], ssem.at[b],
                         add=True)

    def _quad(k, _):
        for j in range(4):
            _chunk(4 * k + j, j)
        return 0

    lax.fori_loop(0, NCHUNK // 4, _quad, 0)
    _chunk(jnp.int32(cl), cl % 4)

    pltpu.make_async_copy(
        rows_v.at[3], acc_sh.at[dst_v.at[cl - 1]], ssem.at[3]).wait()
    pltpu.make_async_copy(
        rows_v.at[0], acc_sh.at[dst_v.at[cl]], ssem.at[0]).wait()

    plsc.subcore_barrier()

    pltpu.sync_copy(acc_sh.at[pl.ds(sid * RPT, RPT)], buf_v)
    pltpu.sync_copy(buf_v, out_hbm.at[cid, pl.ds(sid * RPT, RPT)])

    @pl.when(sid == NS - 1)
    def _():
        pltpu.sync_copy(acc_sh.at[pl.ds(NS * RPT, 16)], buf_v.at[pl.ds(0, 16)])
        pltpu.sync_copy(buf_v.at[pl.ds(0, 16)], out_hbm.at[cid, pl.ds(NS * RPT, 16)])


_seg16 = functools.partial(
    pl.kernel,
    out_type=jax.ShapeDtypeStruct((NC, N, DH), jnp.float32),
    mesh=plsc.VectorSubcoreMesh(core_axis_name="c", subcore_axis_name="s"),
    compiler_params=pltpu.CompilerParams(
        use_tc_tiling_on_sc=False, needs_layout_passes=False),
    scratch_types=[
        pltpu.VMEM((NCHUNK, CH), jnp.int32),
        pltpu.VMEM((NCHUNK, CH), jnp.int32),
        pltpu.VMEM((NCHUNK, CH), jnp.float32),
        pltpu.VMEM((4, CH, DH), jnp.float32),
        pltpu.VMEM((RPT, DH), jnp.float32),
        pltpu.VMEM_SHARED((N, DH), jnp.float32),
        pltpu.SemaphoreType.DMA((4,)),
        pltpu.SemaphoreType.DMA((4,)),
    ],
)(_seg16_body)


def _mid_body(acc_ref, xo_ref, b1_ref, w2_ref, b2_ref, o_ref):
    h = acc_ref[0] + acc_ref[1] + xo_ref[...] + b1_ref[...]
    h = jnp.maximum(h, 0.0)
    o_ref[...] = jnp.dot(h, w2_ref[...], preferred_element_type=jnp.float32) + b2_ref[...]


def _mid(acc, xo, b1, w2, b2):
    return pl.pallas_call(
        _mid_body,
        grid=(10,),
        in_specs=[
            pl.BlockSpec((NC, 1000, DH), lambda i: (0, i, 0)),
            pl.BlockSpec((1000, DH), lambda i: (i, 0)),
            pl.BlockSpec((1, DH), lambda i: (0, 0)),
            pl.BlockSpec((DH, 2), lambda i: (0, 0)),
            pl.BlockSpec((1, 2), lambda i: (0, 0)),
        ],
        out_specs=pl.BlockSpec((1000, 2), lambda i: (i, 0)),
        out_shape=jax.ShapeDtypeStruct((N, 2), jnp.float32),
    )(acc, xo, b1, w2, b2)


def _seg1_body(hr_hbm, src_hbm, dst_hbm, w_hbm, out_hbm,
               hr_v, src_v, dst_v, w_v, prod_v, buf_v, acc_sh, ssem):
    cid = lax.axis_index("c")
    sid = lax.axis_index("s")
    wid = cid * NS + sid

    pltpu.sync_copy(hr_hbm, hr_v)
    pltpu.sync_copy(src_hbm.at[wid], src_v)
    pltpu.sync_copy(dst_hbm.at[wid], dst_v)
    pltpu.sync_copy(w_hbm.at[wid], w_v)

    def _zero(i, _):
        buf_v[pl.ds(i * 16, 16)] = jnp.zeros((16,), jnp.float32)
        return 0
    lax.fori_loop(0, RPT // 16, _zero, 0)
    pltpu.sync_copy(buf_v, acc_sh.at[pl.ds(sid * RPT, RPT)])

    @pl.when(sid == NS - 1)
    def _():
        pltpu.sync_copy(buf_v.at[pl.ds(0, 16)], acc_sh.at[pl.ds(NS * RPT, 16)])

    plsc.subcore_barrier()

    def _fill(prod, c):
        for j in range(CH // 16):
            s_vec = src_v[c, pl.ds(16 * j, 16)]
            vals = plsc.load_gather(hr_v, [s_vec])
            prod[pl.ds(16 * j, 16)] = vals * w_v[c, pl.ds(16 * j, 16)]

    cl = NCHUNK - 1

    def _chunk(c, b):
        @pl.when(c >= 4)
        def _():
            pltpu.make_async_copy(
                prod_v.at[b], acc_sh.at[dst_v.at[c - 4]], ssem.at[b]).wait()
        _fill(prod_v.at[b], c)
        pltpu.async_copy(prod_v.at[b], acc_sh.at[dst_v.at[c]], ssem.at[b],
                         add=True)

    def _quad(k, _):
        for j in range(4):
            _chunk(4 * k + j, j)
        return 0

    lax.fori_loop(0, NCHUNK // 4, _quad, 0)
    _chunk(jnp.int32(cl), cl % 4)

    for c in range(cl - 3, cl + 1):
        pltpu.make_async_copy(
            prod_v.at[c % 4], acc_sh.at[dst_v.at[c]], ssem.at[c % 4]).wait()

    plsc.subcore_barrier()

    pltpu.sync_copy(acc_sh.at[pl.ds(sid * RPT, RPT)], buf_v)
    pltpu.sync_copy(buf_v, out_hbm.at[cid, pl.ds(sid * RPT, RPT)])

    @pl.when(sid == NS - 1)
    def _():
        pltpu.sync_copy(acc_sh.at[pl.ds(NS * RPT, 16)], buf_v.at[pl.ds(0, 16)])
        pltpu.sync_copy(buf_v.at[pl.ds(0, 16)], out_hbm.at[cid, pl.ds(NS * RPT, 16)])


_seg1 = functools.partial(
    pl.kernel,
    out_type=jax.ShapeDtypeStruct((NC, N), jnp.float32),
    mesh=plsc.VectorSubcoreMesh(core_axis_name="c", subcore_axis_name="s"),
    compiler_params=pltpu.CompilerParams(
        use_tc_tiling_on_sc=False, needs_layout_passes=False),
    scratch_types=[
        pltpu.VMEM((N,), jnp.float32),
        pltpu.VMEM((NCHUNK, CH), jnp.int32),
        pltpu.VMEM((NCHUNK, CH), jnp.int32),
        pltpu.VMEM((NCHUNK, CH), jnp.float32),
        pltpu.VMEM((4, CH), jnp.float32),
        pltpu.VMEM((RPT,), jnp.float32),
        pltpu.VMEM_SHARED((N,), jnp.float32),
        pltpu.SemaphoreType.DMA((4,)),
    ],
)(_seg1_body)


def _fin_body(p_ref, hob_ref, o_ref):
    o_ref[...] = p_ref[0:1, :] + p_ref[1:2, :] + hob_ref[...]


def _fin(p, hob):
    return pl.pallas_call(
        _fin_body,
        out_shape=jax.ShapeDtypeStruct((1, N), jnp.float32),
    )(p, hob)


def kernel(x, edge_index, edge_weight, W_rel1, b_rel1, W_root1,
           W_rel2, b_rel2, W_root2):
    src = edge_index[0].astype(jnp.int32).reshape(NW, NCHUNK, CH)
    dst = edge_index[1].astype(jnp.int32).reshape(NW, NCHUNK, CH)
    w3 = edge_weight.reshape(NW, NCHUNK, CH)

    w1t = jnp.concatenate([W_rel1.T, W_root1.T], axis=1)
    w2t = jnp.concatenate([W_rel2.T, W_root2.T], axis=1)
    b2v = jnp.stack([jnp.zeros((), jnp.float32), b_rel2[0]]).reshape(1, 2)

    y_rel, y_root = _proj1(x, w1t)
    acc = _seg16(y_rel, src, dst, w3)
    hx = _mid(acc, y_root, b_rel1.reshape(1, DH), w2t, b2v)
    hr = hx[:, 0]
    hob = hx[:, 1].reshape(1, N)
    p = _seg1(hr, src, dst, w3)
    out = _fin(p, hob)
    return out.reshape(N, 1)


# speedup vs baseline: 29.0395x; 1.1021x over previous
import functools

import jax
import jax.numpy as jnp
from jax import lax
from jax.experimental import pallas as pl
from jax.experimental.pallas import tpu as pltpu
from jax.experimental.pallas import tpu_sc as plsc

N = 10000
E = 320000
DIN = 128
DH = 16

NC = 2
NS = 16
NW = NC * NS
EPW = E // NW
CH = 80
NCHUNK = EPW // CH

RPT = 624


def _proj1_body(x_ref, w_ref, o1_ref, o2_ref):
    r = jnp.dot(x_ref[...], w_ref[...], preferred_element_type=jnp.float32)
    o1_ref[...] = r[:, :DH]
    o2_ref[...] = r[:, DH:]


def _proj1(x, w1t):
    return pl.pallas_call(
        _proj1_body,
        out_shape=[
            jax.ShapeDtypeStruct((N, DH), jnp.float32),
            jax.ShapeDtypeStruct((N, DH), jnp.float32),
        ],
    )(x, w1t)


def _seg16_body(xr_hbm, ei_hbm, w_hbm, out_hbm,
                src_v, dst_v, w_v, rows_v, buf_v, acc_sh, gsem, ssem):
    cid = lax.axis_index("c")
    sid = lax.axis_index("s")
    wid = cid * NS + sid
    ebase = wid * EPW

    pltpu.sync_copy(ei_hbm.at[0, pl.ds(ebase, EPW)], src_v)
    pltpu.sync_copy(ei_hbm.at[1, pl.ds(ebase, EPW)], dst_v)
    pltpu.sync_copy(w_hbm.at[pl.ds(ebase, EPW)], w_v)

    pltpu.async_copy(xr_hbm.at[src_v.at[pl.ds(0, CH)]], rows_v.at[0],
                     gsem.at[0])
    pltpu.async_copy(xr_hbm.at[src_v.at[pl.ds(CH, CH)]], rows_v.at[1],
                     gsem.at[1])

    def _zero(i, _):
        buf_v[i, :] = jnp.zeros((DH,), jnp.float32)
        return 0
    lax.fori_loop(0, RPT, _zero, 0)
    pltpu.sync_copy(buf_v, acc_sh.at[pl.ds(sid * RPT, RPT)])

    @pl.when(sid == NS - 1)
    def _():
        pltpu.sync_copy(buf_v.at[pl.ds(0, 16)], acc_sh.at[pl.ds(NS * RPT, 16)])

    plsc.subcore_barrier()

    def _scale(rows, c):
        for g in range(CH // 16):
            wvec = w_v[pl.ds(c * CH + g * 16, 16)]
            for u in range(16):
                i = g * 16 + u
                rows[i, :] = rows[i, :] * wvec[u]

    cl = NCHUNK - 1

    def _sidx(c):
        return src_v.at[pl.ds(c * CH, CH)]

    def _didx(c):
        return dst_v.at[pl.ds(c * CH, CH)]

    def _chunk(c, b):
        b2 = (b + 2) % 4
        @pl.when(c >= 2)
        def _():
            pltpu.make_async_copy(
                rows_v.at[b2], acc_sh.at[_didx(c - 2)], ssem.at[b2]).wait()
        @pl.when(c + 2 <= cl)
        def _():
            pltpu.async_copy(
                xr_hbm.at[_sidx(c + 2)], rows_v.at[b2], gsem.at[b2])
        pltpu.make_async_copy(
            xr_hbm.at[_sidx(c)], rows_v.at[b], gsem.at[b]).wait()
        _scale(rows_v.at[b], c)
        pltpu.async_copy(rows_v.at[b], acc_sh.at[_didx(c)], ssem.at[b],
                         add=True)

    def _quad(k, _):
        for j in range(4):
            _chunk(4 * k + j, j)
        return 0

    lax.fori_loop(0, NCHUNK // 4, _quad, 0)
    _chunk(jnp.int32(cl), cl % 4)

    pltpu.make_async_copy(
        rows_v.at[3], acc_sh.at[_didx(cl - 1)], ssem.at[3]).wait()
    pltpu.make_async_copy(
        rows_v.at[0], acc_sh.at[_didx(cl)], ssem.at[0]).wait()

    plsc.subcore_barrier()

    pltpu.sync_copy(acc_sh.at[pl.ds(sid * RPT, RPT)], buf_v)
    pltpu.sync_copy(buf_v, out_hbm.at[cid, pl.ds(sid * RPT, RPT)])

    @pl.when(sid == NS - 1)
    def _():
        pltpu.sync_copy(acc_sh.at[pl.ds(NS * RPT, 16)], buf_v.at[pl.ds(0, 16)])
        pltpu.sync_copy(buf_v.at[pl.ds(0, 16)], out_hbm.at[cid, pl.ds(NS * RPT, 16)])


_seg16 = functools.partial(
    pl.kernel,
    out_type=jax.ShapeDtypeStruct((NC, N, DH), jnp.float32),
    mesh=plsc.VectorSubcoreMesh(core_axis_name="c", subcore_axis_name="s"),
    compiler_params=pltpu.CompilerParams(
        use_tc_tiling_on_sc=False, needs_layout_passes=False),
    scratch_types=[
        pltpu.VMEM((EPW,), jnp.int32),
        pltpu.VMEM((EPW,), jnp.int32),
        pltpu.VMEM((EPW,), jnp.float32),
        pltpu.VMEM((4, CH, DH), jnp.float32),
        pltpu.VMEM((RPT, DH), jnp.float32),
        pltpu.VMEM_SHARED((N, DH), jnp.float32),
        pltpu.SemaphoreType.DMA((4,)),
        pltpu.SemaphoreType.DMA((4,)),
    ],
)(_seg16_body)


def _mid_body(acc_ref, xo_ref, b1_ref, w2_ref, b2_ref, o_ref):
    h = acc_ref[0] + acc_ref[1] + xo_ref[...] + b1_ref[...]
    h = jnp.maximum(h, 0.0)
    o_ref[...] = jnp.dot(h, w2_ref[...], preferred_element_type=jnp.float32) + b2_ref[...]


def _mid(acc, xo, b1, w2, b2):
    return pl.pallas_call(
        _mid_body,
        grid=(10,),
        in_specs=[
            pl.BlockSpec((NC, 1000, DH), lambda i: (0, i, 0)),
            pl.BlockSpec((1000, DH), lambda i: (i, 0)),
            pl.BlockSpec((1, DH), lambda i: (0, 0)),
            pl.BlockSpec((DH, 2), lambda i: (0, 0)),
            pl.BlockSpec((1, 2), lambda i: (0, 0)),
        ],
        out_specs=pl.BlockSpec((1000, 2), lambda i: (i, 0)),
        out_shape=jax.ShapeDtypeStruct((N, 2), jnp.float32),
    )(acc, xo, b1, w2, b2)


def _seg1_body(hr_hbm, ei_hbm, w_hbm, out_hbm,
               hr_v, src_v, dst_v, w_v, prod_v, buf_v, acc_sh, ssem):
    cid = lax.axis_index("c")
    sid = lax.axis_index("s")
    wid = cid * NS + sid
    ebase = wid * EPW

    pltpu.sync_copy(hr_hbm, hr_v)
    pltpu.sync_copy(ei_hbm.at[0, pl.ds(ebase, EPW)], src_v)
    pltpu.sync_copy(ei_hbm.at[1, pl.ds(ebase, EPW)], dst_v)
    pltpu.sync_copy(w_hbm.at[pl.ds(ebase, EPW)], w_v)

    def _zero(i, _):
        buf_v[pl.ds(i * 16, 16)] = jnp.zeros((16,), jnp.float32)
        return 0
    lax.fori_loop(0, RPT // 16, _zero, 0)
    pltpu.sync_copy(buf_v, acc_sh.at[pl.ds(sid * RPT, RPT)])

    @pl.when(sid == NS - 1)
    def _():
        pltpu.sync_copy(buf_v.at[pl.ds(0, 16)], acc_sh.at[pl.ds(NS * RPT, 16)])

    plsc.subcore_barrier()

    def _fill(prod, c):
        for j in range(CH // 16):
            s_vec = src_v[pl.ds(c * CH + 16 * j, 16)]
            vals = plsc.load_gather(hr_v, [s_vec])
            prod[pl.ds(16 * j, 16)] = vals * w_v[pl.ds(c * CH + 16 * j, 16)]

    cl = NCHUNK - 1

    def _didx(c):
        return dst_v.at[pl.ds(c * CH, CH)]

    def _chunk(c, b):
        @pl.when(c >= 4)
        def _():
            pltpu.make_async_copy(
                prod_v.at[b], acc_sh.at[_didx(c - 4)], ssem.at[b]).wait()
        _fill(prod_v.at[b], c)
        pltpu.async_copy(prod_v.at[b], acc_sh.at[_didx(c)], ssem.at[b],
                         add=True)

    def _quad(k, _):
        for j in range(4):
            _chunk(4 * k + j, j)
        return 0

    lax.fori_loop(0, NCHUNK // 4, _quad, 0)
    _chunk(jnp.int32(cl), cl % 4)

    for c in range(cl - 3, cl + 1):
        pltpu.make_async_copy(
            prod_v.at[c % 4], acc_sh.at[_didx(c)], ssem.at[c % 4]).wait()

    plsc.subcore_barrier()

    pltpu.sync_copy(acc_sh.at[pl.ds(sid * RPT, RPT)], buf_v)
    pltpu.sync_copy(buf_v, out_hbm.at[cid, pl.ds(sid * RPT, RPT)])

    @pl.when(sid == NS - 1)
    def _():
        pltpu.sync_copy(acc_sh.at[pl.ds(NS * RPT, 16)], buf_v.at[pl.ds(0, 16)])
        pltpu.sync_copy(buf_v.at[pl.ds(0, 16)], out_hbm.at[cid, pl.ds(NS * RPT, 16)])


_seg1 = functools.partial(
    pl.kernel,
    out_type=jax.ShapeDtypeStruct((NC, N), jnp.float32),
    mesh=plsc.VectorSubcoreMesh(core_axis_name="c", subcore_axis_name="s"),
    compiler_params=pltpu.CompilerParams(
        use_tc_tiling_on_sc=False, needs_layout_passes=False),
    scratch_types=[
        pltpu.VMEM((N,), jnp.float32),
        pltpu.VMEM((EPW,), jnp.int32),
        pltpu.VMEM((EPW,), jnp.int32),
        pltpu.VMEM((EPW,), jnp.float32),
        pltpu.VMEM((4, CH), jnp.float32),
        pltpu.VMEM((RPT,), jnp.float32),
        pltpu.VMEM_SHARED((N,), jnp.float32),
        pltpu.SemaphoreType.DMA((4,)),
    ],
)(_seg1_body)


def _fin_body(p_ref, hob_ref, o_ref):
    o_ref[...] = p_ref[0:1, :] + p_ref[1:2, :] + hob_ref[...]


def _fin(p, hob):
    return pl.pallas_call(
        _fin_body,
        out_shape=jax.ShapeDtypeStruct((1, N), jnp.float32),
    )(p, hob)


def kernel(x, edge_index, edge_weight, W_rel1, b_rel1, W_root1,
           W_rel2, b_rel2, W_root2):
    ei = edge_index.astype(jnp.int32)

    w1t = jnp.concatenate([W_rel1.T, W_root1.T], axis=1)
    w2t = jnp.concatenate([W_rel2.T, W_root2.T], axis=1)
    b2v = jnp.stack([jnp.zeros((), jnp.float32), b_rel2[0]]).reshape(1, 2)

    y_rel, y_root = _proj1(x, w1t)
    acc = _seg16(y_rel, ei, edge_weight)
    hx = _mid(acc, y_root, b_rel1.reshape(1, DH), w2t, b2v)
    hr = hx[:, 0]
    hob = hx[:, 1].reshape(1, N)
    p = _seg1(hr, ei, edge_weight)
    out = _fin(p, hob)
    return out.reshape(N, 1)


# speedup vs baseline: 30.8458x; 1.0622x over previous
import functools

import jax
import jax.numpy as jnp
from jax import lax
from jax.experimental import pallas as pl
from jax.experimental.pallas import tpu as pltpu
from jax.experimental.pallas import tpu_sc as plsc

N = 10000
E = 320000
DIN = 128
DH = 16

NC = 2
NS = 16
NW = NC * NS
EPW = E // NW
CH = 80
NCHUNK = EPW // CH

RPT = 624


def _proj1_body(x_ref, w_ref, o1_ref, o2_ref):
    r = jnp.dot(x_ref[...], w_ref[...], preferred_element_type=jnp.float32)
    o1_ref[...] = r[:, :DH]
    o2_ref[...] = r[:, DH:]


def _proj1(x, w1t):
    return pl.pallas_call(
        _proj1_body,
        out_shape=[
            jax.ShapeDtypeStruct((N, DH), jnp.float32),
            jax.ShapeDtypeStruct((N, DH), jnp.float32),
        ],
    )(x, w1t)


def _seg16_body(xr_hbm, src_hbm, dst_hbm, w_hbm, out_hbm,
                src_v, dst_v, w_v, rows_v, buf_v, acc_sh, y_sh, gsem, ssem):
    cid = lax.axis_index("c")
    sid = lax.axis_index("s")
    wid = cid * NS + sid
    ebase = wid * EPW

    pltpu.sync_copy(src_hbm.at[pl.ds(ebase, EPW)], src_v)
    pltpu.sync_copy(dst_hbm.at[pl.ds(ebase, EPW)], dst_v)
    pltpu.sync_copy(w_hbm.at[pl.ds(ebase, EPW)], w_v)

    pltpu.sync_copy(xr_hbm.at[pl.ds(sid * RPT, RPT)], buf_v)
    pltpu.sync_copy(buf_v, y_sh.at[pl.ds(sid * RPT, RPT)])

    @pl.when(sid == NS - 1)
    def _():
        pltpu.sync_copy(xr_hbm.at[pl.ds(NS * RPT, 16)], buf_v.at[pl.ds(0, 16)])
        pltpu.sync_copy(buf_v.at[pl.ds(0, 16)], y_sh.at[pl.ds(NS * RPT, 16)])

    def _zero(i, _):
        buf_v[i, :] = jnp.zeros((DH,), jnp.float32)
        return 0
    lax.fori_loop(0, RPT, _zero, 0)
    pltpu.sync_copy(buf_v, acc_sh.at[pl.ds(sid * RPT, RPT)])

    @pl.when(sid == NS - 1)
    def _():
        pltpu.sync_copy(buf_v.at[pl.ds(0, 16)], acc_sh.at[pl.ds(NS * RPT, 16)])

    plsc.subcore_barrier()

    pltpu.async_copy(y_sh.at[src_v.at[pl.ds(0, CH)]], rows_v.at[0],
                     gsem.at[0])
    pltpu.async_copy(y_sh.at[src_v.at[pl.ds(CH, CH)]], rows_v.at[1],
                     gsem.at[1])

    def _scale(rows, c):
        for g in range(CH // 16):
            wvec = w_v[pl.ds(c * CH + g * 16, 16)]
            for u in range(16):
                i = g * 16 + u
                rows[i, :] = rows[i, :] * wvec[u]

    cl = NCHUNK - 1

    def _sidx(c):
        return src_v.at[pl.ds(c * CH, CH)]

    def _didx(c):
        return dst_v.at[pl.ds(c * CH, CH)]

    def _chunk(c, b):
        b2 = (b + 2) % 4
        @pl.when(c >= 2)
        def _():
            pltpu.make_async_copy(
                rows_v.at[b2], acc_sh.at[_didx(c - 2)], ssem.at[b2]).wait()
        @pl.when(c + 2 <= cl)
        def _():
            pltpu.async_copy(
                y_sh.at[_sidx(c + 2)], rows_v.at[b2], gsem.at[b2])
        pltpu.make_async_copy(
            y_sh.at[_sidx(c)], rows_v.at[b], gsem.at[b]).wait()
        _scale(rows_v.at[b], c)
        pltpu.async_copy(rows_v.at[b], acc_sh.at[_didx(c)], ssem.at[b],
                         add=True)

    def _quad(k, _):
        for j in range(4):
            _chunk(4 * k + j, j)
        return 0

    lax.fori_loop(0, NCHUNK // 4, _quad, 0)
    _chunk(jnp.int32(cl), cl % 4)

    pltpu.make_async_copy(
        rows_v.at[3], acc_sh.at[_didx(cl - 1)], ssem.at[3]).wait()
    pltpu.make_async_copy(
        rows_v.at[0], acc_sh.at[_didx(cl)], ssem.at[0]).wait()

    plsc.subcore_barrier()

    pltpu.sync_copy(acc_sh.at[pl.ds(sid * RPT, RPT)], buf_v)
    pltpu.sync_copy(buf_v, out_hbm.at[cid, pl.ds(sid * RPT, RPT)])

    @pl.when(sid == NS - 1)
    def _():
        pltpu.sync_copy(acc_sh.at[pl.ds(NS * RPT, 16)], buf_v.at[pl.ds(0, 16)])
        pltpu.sync_copy(buf_v.at[pl.ds(0, 16)], out_hbm.at[cid, pl.ds(NS * RPT, 16)])


_seg16 = functools.partial(
    pl.kernel,
    out_type=jax.ShapeDtypeStruct((NC, N, DH), jnp.float32),
    mesh=plsc.VectorSubcoreMesh(core_axis_name="c", subcore_axis_name="s"),
    compiler_params=pltpu.CompilerParams(
        use_tc_tiling_on_sc=False, needs_layout_passes=False),
    scratch_types=[
        pltpu.VMEM((EPW,), jnp.int32),
        pltpu.VMEM((EPW,), jnp.int32),
        pltpu.VMEM((EPW,), jnp.float32),
        pltpu.VMEM((4, CH, DH), jnp.float32),
        pltpu.VMEM((RPT, DH), jnp.float32),
        pltpu.VMEM_SHARED((N, DH), jnp.float32),
        pltpu.VMEM_SHARED((N, DH), jnp.float32),
        pltpu.SemaphoreType.DMA((4,)),
        pltpu.SemaphoreType.DMA((4,)),
    ],
)(_seg16_body)


def _mid_body(acc_ref, xo_ref, b1_ref, w2_ref, b2_ref, o_ref):
    h = acc_ref[0] + acc_ref[1] + xo_ref[...] + b1_ref[...]
    h = jnp.maximum(h, 0.0)
    o_ref[...] = jnp.dot(h, w2_ref[...], preferred_element_type=jnp.float32) + b2_ref[...]


def _mid(acc, xo, b1, w2, b2):
    return pl.pallas_call(
        _mid_body,
        grid=(10,),
        in_specs=[
            pl.BlockSpec((NC, 1000, DH), lambda i: (0, i, 0)),
            pl.BlockSpec((1000, DH), lambda i: (i, 0)),
            pl.BlockSpec((1, DH), lambda i: (0, 0)),
            pl.BlockSpec((DH, 2), lambda i: (0, 0)),
            pl.BlockSpec((1, 2), lambda i: (0, 0)),
        ],
        out_specs=pl.BlockSpec((1000, 2), lambda i: (i, 0)),
        out_shape=jax.ShapeDtypeStruct((N, 2), jnp.float32),
    )(acc, xo, b1, w2, b2)


def _seg1_body(hr_hbm, src_hbm, dst_hbm, w_hbm, out_hbm,
               hr_v, src_v, dst_v, w_v, prod_v, buf_v, acc_sh, ssem):
    cid = lax.axis_index("c")
    sid = lax.axis_index("s")
    wid = cid * NS + sid
    ebase = wid * EPW

    pltpu.sync_copy(hr_hbm, hr_v)
    pltpu.sync_copy(src_hbm.at[pl.ds(ebase, EPW)], src_v)
    pltpu.sync_copy(dst_hbm.at[pl.ds(ebase, EPW)], dst_v)
    pltpu.sync_copy(w_hbm.at[pl.ds(ebase, EPW)], w_v)

    def _zero(i, _):
        buf_v[pl.ds(i * 16, 16)] = jnp.zeros((16,), jnp.float32)
        return 0
    lax.fori_loop(0, RPT // 16, _zero, 0)
    pltpu.sync_copy(buf_v, acc_sh.at[pl.ds(sid * RPT, RPT)])

    @pl.when(sid == NS - 1)
    def _():
        pltpu.sync_copy(buf_v.at[pl.ds(0, 16)], acc_sh.at[pl.ds(NS * RPT, 16)])

    plsc.subcore_barrier()

    def _fill(prod, c):
        for j in range(CH // 16):
            s_vec = src_v[pl.ds(c * CH + 16 * j, 16)]
            vals = plsc.load_gather(hr_v, [s_vec])
            prod[pl.ds(16 * j, 16)] = vals * w_v[pl.ds(c * CH + 16 * j, 16)]

    cl = NCHUNK - 1

    def _didx(c):
        return dst_v.at[pl.ds(c * CH, CH)]

    def _chunk(c, b):
        @pl.when(c >= 4)
        def _():
            pltpu.make_async_copy(
                prod_v.at[b], acc_sh.at[_didx(c - 4)], ssem.at[b]).wait()
        _fill(prod_v.at[b], c)
        pltpu.async_copy(prod_v.at[b], acc_sh.at[_didx(c)], ssem.at[b],
                         add=True)

    def _quad(k, _):
        for j in range(4):
            _chunk(4 * k + j, j)
        return 0

    lax.fori_loop(0, NCHUNK // 4, _quad, 0)
    _chunk(jnp.int32(cl), cl % 4)

    for c in range(cl - 3, cl + 1):
        pltpu.make_async_copy(
            prod_v.at[c % 4], acc_sh.at[_didx(c)], ssem.at[c % 4]).wait()

    plsc.subcore_barrier()

    pltpu.sync_copy(acc_sh.at[pl.ds(sid * RPT, RPT)], buf_v)
    pltpu.sync_copy(buf_v, out_hbm.at[cid, pl.ds(sid * RPT, RPT)])

    @pl.when(sid == NS - 1)
    def _():
        pltpu.sync_copy(acc_sh.at[pl.ds(NS * RPT, 16)], buf_v.at[pl.ds(0, 16)])
        pltpu.sync_copy(buf_v.at[pl.ds(0, 16)], out_hbm.at[cid, pl.ds(NS * RPT, 16)])


_seg1 = functools.partial(
    pl.kernel,
    out_type=jax.ShapeDtypeStruct((NC, N), jnp.float32),
    mesh=plsc.VectorSubcoreMesh(core_axis_name="c", subcore_axis_name="s"),
    compiler_params=pltpu.CompilerParams(
        use_tc_tiling_on_sc=False, needs_layout_passes=False),
    scratch_types=[
        pltpu.VMEM((N,), jnp.float32),
        pltpu.VMEM((EPW,), jnp.int32),
        pltpu.VMEM((EPW,), jnp.int32),
        pltpu.VMEM((EPW,), jnp.float32),
        pltpu.VMEM((4, CH), jnp.float32),
        pltpu.VMEM((RPT,), jnp.float32),
        pltpu.VMEM_SHARED((N,), jnp.float32),
        pltpu.SemaphoreType.DMA((4,)),
    ],
)(_seg1_body)


def _fin_body(p_ref, hob_ref, o_ref):
    o_ref[...] = p_ref[0:1, :] + p_ref[1:2, :] + hob_ref[...]


def _fin(p, hob):
    return pl.pallas_call(
        _fin_body,
        out_shape=jax.ShapeDtypeStruct((1, N), jnp.float32),
    )(p, hob)


def kernel(x, edge_index, edge_weight, W_rel1, b_rel1, W_root1,
           W_rel2, b_rel2, W_root2):
    src = edge_index[0].astype(jnp.int32)
    dst = edge_index[1].astype(jnp.int32)

    w1t = jnp.concatenate([W_rel1.T, W_root1.T], axis=1)
    w2t = jnp.concatenate([W_rel2.T, W_root2.T], axis=1)
    b2v = jnp.stack([jnp.zeros((), jnp.float32), b_rel2[0]]).reshape(1, 2)

    y_rel, y_root = _proj1(x, w1t)
    acc = _seg16(y_rel, src, dst, edge_weight)
    hx = _mid(acc, y_root, b_rel1.reshape(1, DH), w2t, b2v)
    hr = hx[:, 0]
    hob = hx[:, 1].reshape(1, N)
    p = _seg1(hr, src, dst, edge_weight)
    out = _fin(p, hob)
    return out.reshape(N, 1)


# speedup vs baseline: 39.7177x; 1.2876x over previous
import functools

import jax
import jax.numpy as jnp
from jax import lax
from jax.experimental import pallas as pl
from jax.experimental.pallas import tpu as pltpu
from jax.experimental.pallas import tpu_sc as plsc

N = 10000
E = 320000
DIN = 128
DH = 16

NC = 2
NS = 16
NW = NC * NS
EPW = E // NW
CH = 80
NCHUNK = EPW // CH

RPT = 624


def _proj1_body(x_ref, w_ref, o1_ref, o2_ref):
    r = jnp.dot(x_ref[...], w_ref[...], preferred_element_type=jnp.float32)
    o1_ref[...] = r[:, :DH]
    o2_ref[...] = r[:, DH:]


def _proj1(x, w1t):
    return pl.pallas_call(
        _proj1_body,
        out_shape=[
            jax.ShapeDtypeStruct((N, DH), jnp.float32),
            jax.ShapeDtypeStruct((N, DH), jnp.float32),
        ],
    )(x, w1t)


def _seg16_body(xr_hbm, ei_hbm, w_hbm, out_hbm,
                src_v, dst_v, w_v, rows_v, buf_v, acc_sh, y_sh, gsem, ssem):
    cid = lax.axis_index("c")
    sid = lax.axis_index("s")
    wid = cid * NS + sid
    ebase = wid * EPW

    pltpu.sync_copy(ei_hbm.at[0, pl.ds(ebase, EPW)], src_v)
    pltpu.sync_copy(ei_hbm.at[1, pl.ds(ebase, EPW)], dst_v)
    pltpu.sync_copy(w_hbm.at[pl.ds(ebase, EPW)], w_v)

    pltpu.sync_copy(xr_hbm.at[pl.ds(sid * RPT, RPT)], buf_v)
    pltpu.sync_copy(buf_v, y_sh.at[pl.ds(sid * RPT, RPT)])

    @pl.when(sid == NS - 1)
    def _():
        pltpu.sync_copy(xr_hbm.at[pl.ds(NS * RPT, 16)], buf_v.at[pl.ds(0, 16)])
        pltpu.sync_copy(buf_v.at[pl.ds(0, 16)], y_sh.at[pl.ds(NS * RPT, 16)])

    def _zero(i, _):
        buf_v[i, :] = jnp.zeros((DH,), jnp.float32)
        return 0
    lax.fori_loop(0, RPT, _zero, 0)
    pltpu.sync_copy(buf_v, acc_sh.at[pl.ds(sid * RPT, RPT)])

    @pl.when(sid == NS - 1)
    def _():
        pltpu.sync_copy(buf_v.at[pl.ds(0, 16)], acc_sh.at[pl.ds(NS * RPT, 16)])

    plsc.subcore_barrier()

    pltpu.async_copy(y_sh.at[src_v.at[pl.ds(0, CH)]], rows_v.at[0],
                     gsem.at[0])
    pltpu.async_copy(y_sh.at[src_v.at[pl.ds(CH, CH)]], rows_v.at[1],
                     gsem.at[1])

    def _scale(rows, c):
        for g in range(CH // 16):
            wvec = w_v[pl.ds(c * CH + g * 16, 16)]
            for u in range(16):
                i = g * 16 + u
                rows[i, :] = rows[i, :] * wvec[u]

    cl = NCHUNK - 1

    def _sidx(c):
        return src_v.at[pl.ds(c * CH, CH)]

    def _didx(c):
        return dst_v.at[pl.ds(c * CH, CH)]

    def _chunk(c, b):
        b2 = (b + 2) % 4
        @pl.when(c >= 2)
        def _():
            pltpu.make_async_copy(
                rows_v.at[b2], acc_sh.at[_didx(c - 2)], ssem.at[b2]).wait()
        @pl.when(c + 2 <= cl)
        def _():
            pltpu.async_copy(
                y_sh.at[_sidx(c + 2)], rows_v.at[b2], gsem.at[b2])
        pltpu.make_async_copy(
            y_sh.at[_sidx(c)], rows_v.at[b], gsem.at[b]).wait()
        _scale(rows_v.at[b], c)
        pltpu.async_copy(rows_v.at[b], acc_sh.at[_didx(c)], ssem.at[b],
                         add=True)

    def _quad(k, _):
        for j in range(4):
            _chunk(4 * k + j, j)
        return 0

    lax.fori_loop(0, NCHUNK // 4, _quad, 0)
    _chunk(jnp.int32(cl), cl % 4)

    pltpu.make_async_copy(
        rows_v.at[3], acc_sh.at[_didx(cl - 1)], ssem.at[3]).wait()
    pltpu.make_async_copy(
        rows_v.at[0], acc_sh.at[_didx(cl)], ssem.at[0]).wait()

    plsc.subcore_barrier()

    pltpu.sync_copy(acc_sh.at[pl.ds(sid * RPT, RPT)], buf_v)
    pltpu.sync_copy(buf_v, out_hbm.at[cid, pl.ds(sid * RPT, RPT)])

    @pl.when(sid == NS - 1)
    def _():
        pltpu.sync_copy(acc_sh.at[pl.ds(NS * RPT, 16)], buf_v.at[pl.ds(0, 16)])
        pltpu.sync_copy(buf_v.at[pl.ds(0, 16)], out_hbm.at[cid, pl.ds(NS * RPT, 16)])


_seg16 = functools.partial(
    pl.kernel,
    out_type=jax.ShapeDtypeStruct((NC, N, DH), jnp.float32),
    mesh=plsc.VectorSubcoreMesh(core_axis_name="c", subcore_axis_name="s"),
    compiler_params=pltpu.CompilerParams(
        use_tc_tiling_on_sc=False, needs_layout_passes=False),
    scratch_types=[
        pltpu.VMEM((EPW,), jnp.int32),
        pltpu.VMEM((EPW,), jnp.int32),
        pltpu.VMEM((EPW,), jnp.float32),
        pltpu.VMEM((4, CH, DH), jnp.float32),
        pltpu.VMEM((RPT, DH), jnp.float32),
        pltpu.VMEM_SHARED((N, DH), jnp.float32),
        pltpu.VMEM_SHARED((N, DH), jnp.float32),
        pltpu.SemaphoreType.DMA((4,)),
        pltpu.SemaphoreType.DMA((4,)),
    ],
)(_seg16_body)


def _seg1_body(yr_hbm, acc_hbm, ei_hbm, w_hbm, par_hbm, out_hbm,
               hr_v, src_v, dst_v, w_v, prod_v, buf_v, zbuf_v,
               a0_v, a1_v, yr_v, hob_v, ht_v, hrt_v, par_v,
               acc_sh, hr_sh, ssem):
    cid = lax.axis_index("c")
    sid = lax.axis_index("s")
    wid = cid * NS + sid
    ebase = wid * EPW
    nbase = sid * RPT

    pltpu.sync_copy(ei_hbm.at[0, pl.ds(ebase, EPW)], src_v)
    pltpu.sync_copy(ei_hbm.at[1, pl.ds(ebase, EPW)], dst_v)
    pltpu.sync_copy(w_hbm.at[pl.ds(ebase, EPW)], w_v)
    pltpu.sync_copy(par_hbm, par_v)
    pltpu.sync_copy(acc_hbm.at[0, pl.ds(nbase, RPT)], a0_v)
    pltpu.sync_copy(acc_hbm.at[1, pl.ds(nbase, RPT)], a1_v)
    pltpu.sync_copy(yr_hbm.at[pl.ds(nbase, RPT)], yr_v)

    b1 = par_v[0, :]
    wr2 = par_v[1, :]
    wo2 = par_v[2, :]
    idx0 = lax.iota(jnp.int32, 16) * 16

    def _grp(g, _):
        base = g * 16
        for u in range(16):
            h = jnp.maximum(
                a0_v[base + u, :] + a1_v[base + u, :]
                + yr_v[base + u, :] + b1, 0.0)
            ht_v[pl.ds(u * 16, 16)] = h
        hr = jnp.zeros((16,), jnp.float32)
        ho = jnp.zeros((16,), jnp.float32)
        for f in range(16):
            col = plsc.load_gather(ht_v, [idx0 + f])
            hr = hr + col * wr2[f]
            ho = ho + col * wo2[f]
        buf_v[pl.ds(base, 16)] = hr
        hob_v[pl.ds(base, 16)] = ho
        return 0

    lax.fori_loop(0, RPT // 16, _grp, 0)
    pltpu.sync_copy(buf_v, hr_sh.at[pl.ds(nbase, RPT)])

    @pl.when(sid == NS - 1)
    def _():
        pltpu.sync_copy(acc_hbm.at[0, pl.ds(NS * RPT, 16)],
                        a0_v.at[pl.ds(0, 16)])
        pltpu.sync_copy(acc_hbm.at[1, pl.ds(NS * RPT, 16)],
                        a1_v.at[pl.ds(0, 16)])
        pltpu.sync_copy(yr_hbm.at[pl.ds(NS * RPT, 16)], yr_v.at[pl.ds(0, 16)])
        for u in range(16):
            h = jnp.maximum(
                a0_v[u, :] + a1_v[u, :] + yr_v[u, :] + b1, 0.0)
            ht_v[pl.ds(u * 16, 16)] = h
        hr = jnp.zeros((16,), jnp.float32)
        ho = jnp.zeros((16,), jnp.float32)
        for f in range(16):
            col = plsc.load_gather(ht_v, [idx0 + f])
            hr = hr + col * wr2[f]
            ho = ho + col * wo2[f]
        hrt_v[...] = hr
        hob_v[pl.ds(RPT, 16)] = ho
        pltpu.sync_copy(hrt_v, hr_sh.at[pl.ds(NS * RPT, 16)])

    def _zero(i, _):
        zbuf_v[pl.ds(i * 16, 16)] = jnp.zeros((16,), jnp.float32)
        return 0
    lax.fori_loop(0, RPT // 16, _zero, 0)
    pltpu.sync_copy(zbuf_v, acc_sh.at[pl.ds(nbase, RPT)])

    @pl.when(sid == NS - 1)
    def _():
        pltpu.sync_copy(zbuf_v.at[pl.ds(0, 16)], acc_sh.at[pl.ds(NS * RPT, 16)])

    plsc.subcore_barrier()

    pltpu.sync_copy(hr_sh, hr_v)

    def _fill(prod, c):
        for j in range(CH // 16):
            s_vec = src_v[pl.ds(c * CH + 16 * j, 16)]
            vals = plsc.load_gather(hr_v, [s_vec])
            prod[pl.ds(16 * j, 16)] = vals * w_v[pl.ds(c * CH + 16 * j, 16)]

    cl = NCHUNK - 1

    def _didx(c):
        return dst_v.at[pl.ds(c * CH, CH)]

    def _chunk(c, b):
        @pl.when(c >= 4)
        def _():
            pltpu.make_async_copy(
                prod_v.at[b], acc_sh.at[_didx(c - 4)], ssem.at[b]).wait()
        _fill(prod_v.at[b], c)
        pltpu.async_copy(prod_v.at[b], acc_sh.at[_didx(c)], ssem.at[b],
                         add=True)

    def _quad(k, _):
        for j in range(4):
            _chunk(4 * k + j, j)
        return 0

    lax.fori_loop(0, NCHUNK // 4, _quad, 0)
    _chunk(jnp.int32(cl), cl % 4)

    for c in range(cl - 3, cl + 1):
        pltpu.make_async_copy(
            prod_v.at[c % 4], acc_sh.at[_didx(c)], ssem.at[c % 4]).wait()

    plsc.subcore_barrier()

    s = jnp.where(cid == 0, jnp.float32(1.0), jnp.float32(0.0))
    b2v = par_v[3, :]
    pltpu.sync_copy(acc_sh.at[pl.ds(nbase, RPT)], zbuf_v)

    def _fing(g, _):
        v = zbuf_v[pl.ds(g * 16, 16)]
        zbuf_v[pl.ds(g * 16, 16)] = v + s * (hob_v[pl.ds(g * 16, 16)] + b2v)
        return 0
    lax.fori_loop(0, RPT // 16, _fing, 0)
    pltpu.sync_copy(zbuf_v, out_hbm.at[cid, pl.ds(nbase, RPT)])

    @pl.when(sid == NS - 1)
    def _():
        pltpu.sync_copy(acc_sh.at[pl.ds(NS * RPT, 16)], hrt_v)
        hrt_v[...] = hrt_v[...] + s * (hob_v[pl.ds(RPT, 16)] + b2v)
        pltpu.sync_copy(hrt_v, out_hbm.at[cid, pl.ds(NS * RPT, 16)])


_seg1 = functools.partial(
    pl.kernel,
    out_type=jax.ShapeDtypeStruct((NC, N), jnp.float32),
    mesh=plsc.VectorSubcoreMesh(core_axis_name="c", subcore_axis_name="s"),
    compiler_params=pltpu.CompilerParams(
        use_tc_tiling_on_sc=False, needs_layout_passes=False),
    scratch_types=[
        pltpu.VMEM((N,), jnp.float32),
        pltpu.VMEM((EPW,), jnp.int32),
        pltpu.VMEM((EPW,), jnp.int32),
        pltpu.VMEM((EPW,), jnp.float32),
        pltpu.VMEM((4, CH), jnp.float32),
        pltpu.VMEM((RPT,), jnp.float32),
        pltpu.VMEM((RPT,), jnp.float32),
        pltpu.VMEM((RPT, DH), jnp.float32),
        pltpu.VMEM((RPT, DH), jnp.float32),
        pltpu.VMEM((RPT, DH), jnp.float32),
        pltpu.VMEM((RPT + 16,), jnp.float32),
        pltpu.VMEM((256,), jnp.float32),
        pltpu.VMEM((16,), jnp.float32),
        pltpu.VMEM((4, DH), jnp.float32),
        pltpu.VMEM_SHARED((N,), jnp.float32),
        pltpu.VMEM_SHARED((N,), jnp.float32),
        pltpu.SemaphoreType.DMA((4,)),
    ],
)(_seg1_body)


def _fin_body(p_ref, o_ref):
    o_ref[...] = p_ref[0:1, :] + p_ref[1:2, :]


def _fin(p):
    return pl.pallas_call(
        _fin_body,
        out_shape=jax.ShapeDtypeStruct((1, N), jnp.float32),
    )(p)


def kernel(x, edge_index, edge_weight, W_rel1, b_rel1, W_root1,
           W_rel2, b_rel2, W_root2):
    ei = edge_index.astype(jnp.int32)

    w1t = jnp.concatenate([W_rel1.T, W_root1.T], axis=1)
    par = jnp.stack([b_rel1, W_rel2[0], W_root2[0],
                     jnp.full((DH,), b_rel2[0], jnp.float32)])

    y_rel, y_root = _proj1(x, w1t)
    acc = _seg16(y_rel, ei, edge_weight)
    p = _seg1(y_root, acc, ei, edge_weight, par)
    out = _fin(p)
    return out.reshape(N, 1)


# speedup vs baseline: 42.2645x; 1.0641x over previous
import functools

import jax
import jax.numpy as jnp
from jax import lax
from jax.experimental import pallas as pl
from jax.experimental.pallas import tpu as pltpu
from jax.experimental.pallas import tpu_sc as plsc

N = 10000
E = 320000
DIN = 128
DH = 16

NC = 2
NS = 16
NW = NC * NS
EPW = E // NW
CH = 80
NCHUNK = EPW // CH

RPT = 624


def _proj1_body(x_ref, w_ref, o_ref):
    o_ref[...] = jnp.dot(x_ref[...], w_ref[...],
                         preferred_element_type=jnp.float32)


def _proj1(x, w1t):
    return pl.pallas_call(
        _proj1_body,
        out_shape=jax.ShapeDtypeStruct((N, 2 * DH), jnp.float32),
    )(x, w1t)


def _seg16_body(y_hbm, ei_hbm, w_hbm, out_hbm,
                src_v, dst_v, w_v, rows_v, buf_v, yb_v, acc_sh, y_sh,
                gsem, ssem):
    cid = lax.axis_index("c")
    sid = lax.axis_index("s")
    wid = cid * NS + sid
    ebase = wid * EPW
    nbase = sid * RPT

    pltpu.async_copy(ei_hbm.at[0, pl.ds(ebase, EPW)], src_v, ssem.at[1])
    pltpu.async_copy(ei_hbm.at[1, pl.ds(ebase, EPW)], dst_v, ssem.at[2])
    pltpu.async_copy(w_hbm.at[pl.ds(ebase, EPW)], w_v, ssem.at[3])
    pltpu.async_copy(y_hbm.at[pl.ds(nbase, RPT)], yb_v, gsem.at[2])

    pltpu.make_async_copy(y_hbm.at[pl.ds(nbase, RPT)], yb_v, gsem.at[2]).wait()

    def _xtr(g, _):
        for u in range(8):
            i = g * 8 + u
            buf_v[i, :] = yb_v[i, pl.ds(0, DH)]
        return 0
    lax.fori_loop(0, RPT // 8, _xtr, 0)
    pltpu.sync_copy(buf_v, y_sh.at[pl.ds(nbase, RPT)])

    @pl.when(sid == NS - 1)
    def _():
        pltpu.sync_copy(y_hbm.at[pl.ds(NS * RPT, 16)], yb_v.at[pl.ds(0, 16)])
        for i in range(16):
            buf_v[i, :] = yb_v[i, pl.ds(0, DH)]
        pltpu.sync_copy(buf_v.at[pl.ds(0, 16)], y_sh.at[pl.ds(NS * RPT, 16)])

    def _zero(g, _):
        for u in range(8):
            buf_v[g * 8 + u, :] = jnp.zeros((DH,), jnp.float32)
        return 0
    lax.fori_loop(0, RPT // 8, _zero, 0)
    pltpu.sync_copy(buf_v, acc_sh.at[pl.ds(nbase, RPT)])

    @pl.when(sid == NS - 1)
    def _():
        pltpu.sync_copy(buf_v.at[pl.ds(0, 16)], acc_sh.at[pl.ds(NS * RPT, 16)])

    pltpu.make_async_copy(ei_hbm.at[0, pl.ds(ebase, EPW)], src_v,
                          ssem.at[1]).wait()
    pltpu.make_async_copy(ei_hbm.at[1, pl.ds(ebase, EPW)], dst_v,
                          ssem.at[2]).wait()
    pltpu.make_async_copy(w_hbm.at[pl.ds(ebase, EPW)], w_v, ssem.at[3]).wait()

    plsc.subcore_barrier()

    pltpu.async_copy(y_sh.at[src_v.at[pl.ds(0, CH)]], rows_v.at[0],
                     gsem.at[0])
    pltpu.async_copy(y_sh.at[src_v.at[pl.ds(CH, CH)]], rows_v.at[1],
                     gsem.at[1])

    def _scale(rows, c):
        for g in range(CH // 16):
            wvec = w_v[pl.ds(c * CH + g * 16, 16)]
            for u in range(16):
                i = g * 16 + u
                rows[i, :] = rows[i, :] * wvec[u]

    cl = NCHUNK - 1

    def _sidx(c):
        return src_v.at[pl.ds(c * CH, CH)]

    def _didx(c):
        return dst_v.at[pl.ds(c * CH, CH)]

    def _chunk(c, b):
        b2 = (b + 2) % 4
        @pl.when(c >= 2)
        def _():
            pltpu.make_async_copy(
                rows_v.at[b2], acc_sh.at[_didx(c - 2)], ssem.at[b2]).wait()
        @pl.when(c + 2 <= cl)
        def _():
            pltpu.async_copy(
                y_sh.at[_sidx(c + 2)], rows_v.at[b2], gsem.at[b2])
        pltpu.make_async_copy(
            y_sh.at[_sidx(c)], rows_v.at[b], gsem.at[b]).wait()
        _scale(rows_v.at[b], c)
        pltpu.async_copy(rows_v.at[b], acc_sh.at[_didx(c)], ssem.at[b],
                         add=True)

    def _quad(k, _):
        for j in range(4):
            _chunk(4 * k + j, j)
        return 0

    lax.fori_loop(0, NCHUNK // 4, _quad, 0)
    _chunk(jnp.int32(cl), cl % 4)

    pltpu.make_async_copy(
        rows_v.at[3], acc_sh.at[_didx(cl - 1)], ssem.at[3]).wait()
    pltpu.make_async_copy(
        rows_v.at[0], acc_sh.at[_didx(cl)], ssem.at[0]).wait()

    plsc.subcore_barrier()

    pltpu.sync_copy(acc_sh.at[pl.ds(sid * RPT, RPT)], buf_v)
    pltpu.sync_copy(buf_v, out_hbm.at[cid, pl.ds(sid * RPT, RPT)])

    @pl.when(sid == NS - 1)
    def _():
        pltpu.sync_copy(acc_sh.at[pl.ds(NS * RPT, 16)], buf_v.at[pl.ds(0, 16)])
        pltpu.sync_copy(buf_v.at[pl.ds(0, 16)], out_hbm.at[cid, pl.ds(NS * RPT, 16)])


_seg16 = functools.partial(
    pl.kernel,
    out_type=jax.ShapeDtypeStruct((NC, N, DH), jnp.float32),
    mesh=plsc.VectorSubcoreMesh(core_axis_name="c", subcore_axis_name="s"),
    compiler_params=pltpu.CompilerParams(
        use_tc_tiling_on_sc=False, needs_layout_passes=False),
    scratch_types=[
        pltpu.VMEM((EPW,), jnp.int32),
        pltpu.VMEM((EPW,), jnp.int32),
        pltpu.VMEM((EPW,), jnp.float32),
        pltpu.VMEM((4, CH, DH), jnp.float32),
        pltpu.VMEM((RPT, DH), jnp.float32),
        pltpu.VMEM((RPT, 2 * DH), jnp.float32),
        pltpu.VMEM_SHARED((N, DH), jnp.float32),
        pltpu.VMEM_SHARED((N, DH), jnp.float32),
        pltpu.SemaphoreType.DMA((4,)),
        pltpu.SemaphoreType.DMA((4,)),
    ],
)(_seg16_body)


def _seg1_body(yr_hbm, acc_hbm, ei_hbm, w_hbm, par_hbm, out_hbm,
               hr_v, src_v, dst_v, w_v, prod_v, buf_v, zbuf_v,
               a0_v, a1_v, yr_v, hob_v, ht_v, hrt_v, par_v,
               acc_sh, hr_sh, ssem):
    cid = lax.axis_index("c")
    sid = lax.axis_index("s")
    wid = cid * NS + sid
    ebase = wid * EPW
    nbase = sid * RPT

    pltpu.async_copy(ei_hbm.at[0, pl.ds(ebase, EPW)], src_v, ssem.at[1])
    pltpu.async_copy(ei_hbm.at[1, pl.ds(ebase, EPW)], dst_v, ssem.at[2])
    pltpu.async_copy(w_hbm.at[pl.ds(ebase, EPW)], w_v, ssem.at[3])
    pltpu.sync_copy(par_hbm, par_v)
    pltpu.sync_copy(acc_hbm.at[0, pl.ds(nbase, RPT)], a0_v)
    pltpu.sync_copy(acc_hbm.at[1, pl.ds(nbase, RPT)], a1_v)
    pltpu.sync_copy(yr_hbm.at[pl.ds(nbase, RPT)], yr_v)

    b1 = par_v[0, :]
    wr2 = par_v[1, :]
    wo2 = par_v[2, :]
    idx0 = lax.iota(jnp.int32, 16) * 16

    def _grp(g, _):
        base = g * 16
        for u in range(16):
            h = jnp.maximum(
                a0_v[base + u, :] + a1_v[base + u, :]
                + yr_v[base + u, pl.ds(DH, DH)] + b1, 0.0)
            ht_v[pl.ds(u * 16, 16)] = h
        hr = jnp.zeros((16,), jnp.float32)
        ho = jnp.zeros((16,), jnp.float32)
        for f in range(16):
            col = plsc.load_gather(ht_v, [idx0 + f])
            hr = hr + col * wr2[f]
            ho = ho + col * wo2[f]
        buf_v[pl.ds(base, 16)] = hr
        hob_v[pl.ds(base, 16)] = ho
        return 0

    lax.fori_loop(0, RPT // 16, _grp, 0)
    pltpu.sync_copy(buf_v, hr_sh.at[pl.ds(nbase, RPT)])

    @pl.when(sid == NS - 1)
    def _():
        pltpu.sync_copy(acc_hbm.at[0, pl.ds(NS * RPT, 16)],
                        a0_v.at[pl.ds(0, 16)])
        pltpu.sync_copy(acc_hbm.at[1, pl.ds(NS * RPT, 16)],
                        a1_v.at[pl.ds(0, 16)])
        pltpu.sync_copy(yr_hbm.at[pl.ds(NS * RPT, 16)], yr_v.at[pl.ds(0, 16)])
        for u in range(16):
            h = jnp.maximum(
                a0_v[u, :] + a1_v[u, :] + yr_v[u, pl.ds(DH, DH)] + b1, 0.0)
            ht_v[pl.ds(u * 16, 16)] = h
        hr = jnp.zeros((16,), jnp.float32)
        ho = jnp.zeros((16,), jnp.float32)
        for f in range(16):
            col = plsc.load_gather(ht_v, [idx0 + f])
            hr = hr + col * wr2[f]
            ho = ho + col * wo2[f]
        hrt_v[...] = hr
        hob_v[pl.ds(RPT, 16)] = ho
        pltpu.sync_copy(hrt_v, hr_sh.at[pl.ds(NS * RPT, 16)])

    def _zero(g, _):
        for u in range(4):
            zbuf_v[pl.ds((g * 4 + u) * 16, 16)] = jnp.zeros((16,), jnp.float32)
        return 0
    lax.fori_loop(0, RPT // 64, _zero, 0)
    for u in range(RPT // 64 * 4, RPT // 16):
        zbuf_v[pl.ds(u * 16, 16)] = jnp.zeros((16,), jnp.float32)
    pltpu.sync_copy(zbuf_v, acc_sh.at[pl.ds(nbase, RPT)])

    @pl.when(sid == NS - 1)
    def _():
        pltpu.sync_copy(zbuf_v.at[pl.ds(0, 16)], acc_sh.at[pl.ds(NS * RPT, 16)])

    pltpu.make_async_copy(ei_hbm.at[0, pl.ds(ebase, EPW)], src_v,
                          ssem.at[1]).wait()
    pltpu.make_async_copy(ei_hbm.at[1, pl.ds(ebase, EPW)], dst_v,
                          ssem.at[2]).wait()
    pltpu.make_async_copy(w_hbm.at[pl.ds(ebase, EPW)], w_v, ssem.at[3]).wait()

    plsc.subcore_barrier()

    pltpu.sync_copy(hr_sh, hr_v)

    def _fill(prod, c):
        for j in range(CH // 16):
            s_vec = src_v[pl.ds(c * CH + 16 * j, 16)]
            vals = plsc.load_gather(hr_v, [s_vec])
            prod[pl.ds(16 * j, 16)] = vals * w_v[pl.ds(c * CH + 16 * j, 16)]

    cl = NCHUNK - 1

    def _didx(c):
        return dst_v.at[pl.ds(c * CH, CH)]

    def _chunk(c, b):
        @pl.when(c >= 4)
        def _():
            pltpu.make_async_copy(
                prod_v.at[b], acc_sh.at[_didx(c - 4)], ssem.at[b]).wait()
        _fill(prod_v.at[b], c)
        pltpu.async_copy(prod_v.at[b], acc_sh.at[_didx(c)], ssem.at[b],
                         add=True)

    def _quad(k, _):
        for j in range(4):
            _chunk(4 * k + j, j)
        return 0

    lax.fori_loop(0, NCHUNK // 4, _quad, 0)
    _chunk(jnp.int32(cl), cl % 4)

    for c in range(cl - 3, cl + 1):
        pltpu.make_async_copy(
            prod_v.at[c % 4], acc_sh.at[_didx(c)], ssem.at[c % 4]).wait()

    plsc.subcore_barrier()

    s = jnp.where(cid == 0, jnp.float32(1.0), jnp.float32(0.0))
    b2v = par_v[3, :]
    pltpu.sync_copy(acc_sh.at[pl.ds(nbase, RPT)], zbuf_v)

    def _fing(g, _):
        v = zbuf_v[pl.ds(g * 16, 16)]
        zbuf_v[pl.ds(g * 16, 16)] = v + s * (hob_v[pl.ds(g * 16, 16)] + b2v)
        return 0
    lax.fori_loop(0, RPT // 16, _fing, 0)
    pltpu.sync_copy(zbuf_v, out_hbm.at[cid, pl.ds(nbase, RPT)])

    @pl.when(sid == NS - 1)
    def _():
        pltpu.sync_copy(acc_sh.at[pl.ds(NS * RPT, 16)], hrt_v)
        hrt_v[...] = hrt_v[...] + s * (hob_v[pl.ds(RPT, 16)] + b2v)
        pltpu.sync_copy(hrt_v, out_hbm.at[cid, pl.ds(NS * RPT, 16)])


_seg1 = functools.partial(
    pl.kernel,
    out_type=jax.ShapeDtypeStruct((NC, N), jnp.float32),
    mesh=plsc.VectorSubcoreMesh(core_axis_name="c", subcore_axis_name="s"),
    compiler_params=pltpu.CompilerParams(
        use_tc_tiling_on_sc=False, needs_layout_passes=False),
    scratch_types=[
        pltpu.VMEM((N,), jnp.float32),
        pltpu.VMEM((EPW,), jnp.int32),
        pltpu.VMEM((EPW,), jnp.int32),
        pltpu.VMEM((EPW,), jnp.float32),
        pltpu.VMEM((4, CH), jnp.float32),
        pltpu.VMEM((RPT,), jnp.float32),
        pltpu.VMEM((RPT,), jnp.float32),
        pltpu.VMEM((RPT, DH), jnp.float32),
        pltpu.VMEM((RPT, DH), jnp.float32),
        pltpu.VMEM((RPT, 2 * DH), jnp.float32),
        pltpu.VMEM((RPT + 16,), jnp.float32),
        pltpu.VMEM((256,), jnp.float32),
        pltpu.VMEM((16,), jnp.float32),
        pltpu.VMEM((4, DH), jnp.float32),
        pltpu.VMEM_SHARED((N,), jnp.float32),
        pltpu.VMEM_SHARED((N,), jnp.float32),
        pltpu.SemaphoreType.DMA((4,)),
    ],
)(_seg1_body)


def _fin_body(p_ref, o_ref):
    o_ref[...] = p_ref[0:1, :] + p_ref[1:2, :]


def _fin(p):
    return pl.pallas_call(
        _fin_body,
        out_shape=jax.ShapeDtypeStruct((1, N), jnp.float32),
    )(p)


def kernel(x, edge_index, edge_weight, W_rel1, b_rel1, W_root1,
           W_rel2, b_rel2, W_root2):
    ei = edge_index.astype(jnp.int32)

    w1t = jnp.concatenate([W_rel1.T, W_root1.T], axis=1)
    par = jnp.stack([b_rel1, W_rel2[0], W_root2[0],
                     jnp.full((DH,), b_rel2[0], jnp.float32)])

    y = _proj1(x, w1t)
    acc = _seg16(y, ei, edge_weight)
    p = _seg1(y, acc, ei, edge_weight, par)
    out = _fin(p)
    return out.reshape(N, 1)
